# baseline (speedup 1.0000x reference)
import functools

import jax
import jax.numpy as jnp
from jax import lax
from jax.experimental import pallas as pl
from jax.experimental.pallas import tpu as pltpu
from jax.experimental.pallas import tpu_sc as plsc

B = 4
B_TC = 3
N_P = 8192
D = 768
PB = 1024

NC = 2
NS = 16
NW = NC * NS
ROWS_PER_W = N_P // NW
R = 32
LANES = 16
VECS_PER_ROW = D // LANES

_mesh = plsc.VectorSubcoreMesh(core_axis_name="c", subcore_axis_name="s")


@functools.partial(
    pl.kernel,
    mesh=_mesh,
    out_type=jax.ShapeDtypeStruct((B - B_TC, N_P, D), jnp.float32),
    scratch_types=[
        pltpu.VMEM((R, D), jnp.float32),
        pltpu.VMEM((R, D), jnp.float32),
    ],
)
def _sc_kernel(patches_hbm, pos_hbm, out_hbm, pbuf, abuf):
    wid = lax.axis_index("s") * NC + lax.axis_index("c")
    base = wid * ROWS_PER_W

    def chunk_body(ci, carry):
        rbase = base + ci * R
        pltpu.sync_copy(pos_hbm.at[pl.ds(rbase, R)], pbuf)

        for b in range(B_TC, B):
            pltpu.sync_copy(patches_hbm.at[b, pl.ds(rbase, R)], abuf)

            def row_body(r, c3):
                for j in range(VECS_PER_ROW):
                    sl = pl.ds(j * LANES, LANES)
                    abuf[r, sl] = abuf[r, sl] + pbuf[r, sl]
                return c3

            lax.fori_loop(0, R, row_body, carry)
            pltpu.sync_copy(abuf, out_hbm.at[b - B_TC, pl.ds(rbase, R)])
        return carry

    lax.fori_loop(0, ROWS_PER_W // R, chunk_body, 0)


def _tc_add_kernel(patches_ref, pos_ref, out_ref):
    out_ref[...] = patches_ref[...] + pos_ref[...][None, :, :]


def _tc_part(patches, pos_table):
    return pl.pallas_call(
        _tc_add_kernel,
        grid=(N_P // PB,),
        in_specs=[
            pl.BlockSpec((B_TC, PB, D), lambda i: (0, i, 0)),
            pl.BlockSpec((PB, D), lambda i: (i, 0)),
        ],
        out_specs=pl.BlockSpec((B_TC, PB, D), lambda i: (0, i, 0)),
        out_shape=jax.ShapeDtypeStruct((B_TC, N_P, D), jnp.float32),
    )(patches, pos_table)


def kernel(patches, pos_table):
    sc_out = _sc_kernel(patches, pos_table)
    tc_out = _tc_part(patches, pos_table)
    return jnp.concatenate([tc_out, sc_out], axis=0)


# speedup vs baseline: 2.1208x; 2.1208x over previous
import jax
import jax.numpy as jnp
from jax.experimental import pallas as pl

B = 4
N_P = 8192
D = 768
PB = 1024


def _add_kernel(patches_ref, pos_ref, out_ref):
    out_ref[...] = patches_ref[...] + pos_ref[...]


def kernel(patches, pos_table):
    grid = (N_P // PB, B)
    return pl.pallas_call(
        _add_kernel,
        grid=grid,
        in_specs=[
            pl.BlockSpec((1, PB, D), lambda i, b: (b, i, 0)),
            pl.BlockSpec((PB, D), lambda i, b: (i, 0)),
        ],
        out_specs=pl.BlockSpec((1, PB, D), lambda i, b: (b, i, 0)),
        out_shape=jax.ShapeDtypeStruct((B, N_P, D), jnp.float32),
    )(patches, pos_table)


# speedup vs baseline: 2.2719x; 1.0713x over previous
import jax
import jax.numpy as jnp
from jax.experimental import pallas as pl
from jax.experimental.pallas import tpu as pltpu

B = 4
N_P = 8192
D = 768
PB = 1024


def _add_kernel(patches_ref, pos_ref, out_ref):
    out_ref[...] = patches_ref[...] + pos_ref[...][None, :, :]


def kernel(patches, pos_table):
    grid = (N_P // PB,)
    return pl.pallas_call(
        _add_kernel,
        grid=grid,
        in_specs=[
            pl.BlockSpec((B, PB, D), lambda i: (0, i, 0)),
            pl.BlockSpec((PB, D), lambda i: (i, 0)),
        ],
        out_specs=pl.BlockSpec((B, PB, D), lambda i: (0, i, 0)),
        out_shape=jax.ShapeDtypeStruct((B, N_P, D), jnp.float32),
        compiler_params=pltpu.CompilerParams(dimension_semantics=("parallel",)),
    )(patches, pos_table)


# speedup vs baseline: 2.2728x; 1.0004x over previous
import jax
import jax.numpy as jnp
from jax.experimental import pallas as pl

B = 4
N_P = 8192
D = 768
PB = 1024


def _add_kernel(patches_ref, pos_ref, out_ref):
    out_ref[...] = patches_ref[...] + pos_ref[...][None, :, :]


def kernel(patches, pos_table):
    grid = (N_P // PB,)
    return pl.pallas_call(
        _add_kernel,
        grid=grid,
        in_specs=[
            pl.BlockSpec((B, PB, D), lambda i: (0, i, 0)),
            pl.BlockSpec((PB, D), lambda i: (i, 0)),
        ],
        out_specs=pl.BlockSpec((B, PB, D), lambda i: (0, i, 0)),
        out_shape=jax.ShapeDtypeStruct((B, N_P, D), jnp.float32),
    )(patches, pos_table)
